# baseline (speedup 1.0000x reference)
import functools

import jax
import jax.numpy as jnp
from jax import lax
from jax.experimental import pallas as pl
from jax.experimental.pallas import tpu as pltpu
from jax.experimental.pallas import tpu_sc as plsc

B, L, V, D = 4096, 200, 100000, 64
N = B * L
DH = D // 2
NC, NS = 2, 16
NW = NC * NS
ROWS_PER_W = N // NW
CK = 2 * L
NCH = ROWS_PER_W // CK
LANES = 16


def _sc_embed(x_flat, tok_half, pos_weight):
    mesh = plsc.VectorSubcoreMesh(core_axis_name="c", subcore_axis_name="s")

    @functools.partial(
        pl.kernel,
        mesh=mesh,
        compiler_params=pltpu.CompilerParams(use_tc_tiling_on_sc=False),
        out_type=jax.ShapeDtypeStruct((N, D), jnp.float32),
        scratch_types=(
            [pltpu.VMEM((CK,), jnp.int32) for _ in range(2)]
            + [pltpu.VMEM((CK,), jnp.int32) for _ in range(4)]
            + [pltpu.VMEM((CK, DH), jnp.float32) for _ in range(4)]
            + [pltpu.VMEM((CK, D), jnp.float32) for _ in range(2)]
            + [pltpu.VMEM((L, D), jnp.float32)]
            + [pltpu.SemaphoreType.DMA for _ in range(8)]
        ),
    )
    def k(x_hbm, tok_hbm, pos_hbm, out_hbm, *s):
        raw = s[0:2]
        idxA = s[2:4]
        idxB = s[4:6]
        bufA = s[6:8]
        bufB = s[8:10]
        outb = s[10:12]
        pos_v = s[12]
        si = s[13:15]
        sgA = s[15:17]
        sgB = s[17:19]
        so = s[19:21]

        wid = lax.axis_index("s") * NC + lax.axis_index("c")
        base = wid * ROWS_PER_W
        pltpu.sync_copy(pos_hbm, pos_v)

        def fire_idx(c, b):
            pltpu.async_copy(x_hbm.at[pl.ds(base + c * CK, CK)], raw[b], si[b])

        def wait_idx(b):
            pltpu.make_async_copy(x_hbm.at[pl.ds(0, CK)], raw[b], si[b]).wait()

        def prep_idx(b):
            def body(i, carry):
                sl = pl.ds(i * LANES, LANES)
                v = raw[b][sl]
                v2 = v + v
                idxA[b][sl] = v2
                idxB[b][sl] = v2 + 1
                return carry

            lax.fori_loop(0, CK // LANES, body, 0)

        def fire_gathers(b):
            pltpu.async_copy(tok_hbm.at[idxA[b]], bufA[b], sgA[b])
            pltpu.async_copy(tok_hbm.at[idxB[b]], bufB[b], sgB[b])

        def wait_gathers(b):
            pltpu.make_async_copy(tok_hbm.at[pl.ds(0, CK)], bufA[b], sgA[b]).wait()
            pltpu.make_async_copy(tok_hbm.at[pl.ds(0, CK)], bufB[b], sgB[b]).wait()

        def fire_out(c, b):
            pltpu.async_copy(outb[b], out_hbm.at[pl.ds(base + c * CK, CK)], so[b])

        def wait_out(b):
            pltpu.make_async_copy(out_hbm.at[pl.ds(0, CK)], outb[b], so[b]).wait()

        def merge_add(b):
            a_ref, b_ref, o_ref = bufA[b], bufB[b], outb[b]

            def row_body(r, carry):
                for rep in range(CK // L):
                    row = rep * L + r
                    for kk in range(DH // LANES):
                        sl = pl.ds(kk * LANES, LANES)
                        slo = pl.ds(kk * LANES, LANES)
                        shi = pl.ds(DH + kk * LANES, LANES)
                        o_ref[row, slo] = a_ref[row, sl] + pos_v[r, slo]
                        o_ref[row, shi] = b_ref[row, sl] + pos_v[r, shi]
                return carry

            lax.fori_loop(0, L, row_body, 0)

        fire_idx(0, 0)
        wait_idx(0)
        prep_idx(0)
        fire_gathers(0)
        fire_idx(1, 1)

        def pair_body(p, carry):
            for b in range(2):
                c = 2 * p + b
                nb = 1 - b
                wait_gathers(b)

                @pl.when(c + 1 < NCH)
                def _():
                    wait_idx(nb)
                    prep_idx(nb)
                    fire_gathers(nb)

                    @pl.when(c + 2 < NCH)
                    def _():
                        fire_idx(c + 2, b)

                @pl.when(c >= 2)
                def _():
                    wait_out(b)

                merge_add(b)
                fire_out(c, b)
            return carry

        lax.fori_loop(0, NCH // 2, pair_body, 0)
        wait_out(0)
        wait_out(1)

    return k(x_flat, tok_half, pos_weight)


def kernel(x, token_weight, pos_weight):
    x_flat = x.reshape(-1).astype(jnp.int32)
    tok_half = token_weight.reshape(2 * V, DH)
    out = _sc_embed(x_flat, tok_half, pos_weight)
    return out.reshape(B, L, D)


# speedup vs baseline: 1.5239x; 1.5239x over previous
import functools

import jax
import jax.numpy as jnp
from jax import lax
from jax.experimental import pallas as pl
from jax.experimental.pallas import tpu as pltpu
from jax.experimental.pallas import tpu_sc as plsc

B, L, V, D = 4096, 200, 100000, 64
N = B * L
NC, NS = 2, 16
NW = NC * NS
ROWS_PER_W = N // NW
CK = 2 * L
NCH = ROWS_PER_W // CK
LANES = 16


def _sc_embed(x_flat, token_weight, pos_weight):
    mesh = plsc.VectorSubcoreMesh(core_axis_name="c", subcore_axis_name="s")

    @functools.partial(
        pl.kernel,
        mesh=mesh,
        compiler_params=pltpu.CompilerParams(use_tc_tiling_on_sc=False),
        out_type=jax.ShapeDtypeStruct((N, D), jnp.float32),
        scratch_types=(
            [pltpu.VMEM((CK,), jnp.int32) for _ in range(2)]
            + [pltpu.VMEM((CK, D), jnp.float32) for _ in range(2)]
            + [pltpu.VMEM((L, D), jnp.float32)]
            + [pltpu.SemaphoreType.DMA for _ in range(6)]
        ),
    )
    def k(x_hbm, tok_hbm, pos_hbm, out_hbm, *s):
        idx_b = s[0:2]
        rows_b = s[2:4]
        pos_v = s[4]
        si = s[5:7]
        sg = s[7:9]
        so = s[9:11]

        wid = lax.axis_index("s") * NC + lax.axis_index("c")
        base = wid * ROWS_PER_W
        pltpu.sync_copy(pos_hbm, pos_v)

        def fire_idx(c, b):
            pltpu.async_copy(x_hbm.at[pl.ds(base + c * CK, CK)], idx_b[b], si[b])

        def wait_idx(b):
            pltpu.make_async_copy(x_hbm.at[pl.ds(0, CK)], idx_b[b], si[b]).wait()

        def fire_gather(b):
            pltpu.async_copy(tok_hbm.at[idx_b[b]], rows_b[b], sg[b])

        def wait_gather(b):
            pltpu.make_async_copy(tok_hbm.at[pl.ds(0, CK)], rows_b[b],
                                  sg[b]).wait()

        def fire_out(c, b):
            pltpu.async_copy(rows_b[b], out_hbm.at[pl.ds(base + c * CK, CK)],
                             so[b])

        def wait_out(b):
            pltpu.make_async_copy(out_hbm.at[pl.ds(0, CK)], rows_b[b],
                                  so[b]).wait()

        def add_pos(b):
            rows = rows_b[b]

            def row_body(r, carry):
                for rep in range(CK // L):
                    row = rep * L + r
                    for kk in range(D // LANES):
                        sl = pl.ds(kk * LANES, LANES)
                        rows[row, sl] = rows[row, sl] + pos_v[r, sl]
                return carry

            lax.fori_loop(0, L, row_body, 0)

        fire_idx(0, 0)
        wait_idx(0)
        fire_gather(0)
        fire_idx(1, 1)

        def pair_body(p, carry):
            for b in range(2):
                c = 2 * p + b
                nb = 1 - b
                wait_gather(b)

                @pl.when(c + 1 < NCH)
                def _():
                    wait_idx(nb)

                    @pl.when(c >= 1)
                    def _():
                        wait_out(nb)

                    fire_gather(nb)

                    @pl.when(c + 2 < NCH)
                    def _():
                        fire_idx(c + 2, b)

                add_pos(b)
                fire_out(c, b)
            return carry

        lax.fori_loop(0, NCH // 2, pair_body, 0)
        wait_out(0)
        wait_out(1)

    return k(x_flat, token_weight, pos_weight)


def kernel(x, token_weight, pos_weight):
    x_flat = x.reshape(-1).astype(jnp.int32)
    out = _sc_embed(x_flat, token_weight, pos_weight)
    return out.reshape(B, L, D)
